# baseline (speedup 1.0000x reference)
import functools

import jax
import jax.numpy as jnp
from jax import lax
from jax.experimental import pallas as pl
from jax.experimental.pallas import tpu as pltpu
from jax.experimental.pallas import tpu_sc as plsc

H = 8
D = 64
HD = H * D

NC = 2
NS = 16
NW = NC * NS
CHS = 296

CH = 512
TT = 128


def _sc_gather_body(feats_hbm, idx_hbm, out,
                    b0, b1, b2, ib0, ib1, ib2, sem,
                    *, ch, k, e_pad):
    wid = lax.axis_index("s") * NC + lax.axis_index("c")
    ibs = (ib0, ib1, ib2)
    bufs = (b0, b1, b2)

    def jbody(j, _):
        row0 = wid * ch + j * CHS
        for l in range(3):
            pltpu.sync_copy(idx_hbm.at[pl.ds(l * e_pad + row0, CHS)], ibs[l])
        cps = [pltpu.async_copy(feats_hbm.at[ibs[l]], bufs[l], sem)
               for l in range(3)]
        for c in cps:
            c.wait()
        for l in range(3):
            pltpu.sync_copy(bufs[l],
                            out.at[pl.ds(row0, CHS), pl.ds(l * D, D)])
        return 0

    lax.fori_loop(0, k, jbody, 0)


def _sc_gather(feats, idxs, e_pad):
    ch = e_pad // NW
    k = ch // CHS
    mesh = plsc.VectorSubcoreMesh(core_axis_name="c", subcore_axis_name="s")
    fn = pl.kernel(
        functools.partial(_sc_gather_body, ch=ch, k=k, e_pad=e_pad),
        out_type=jax.ShapeDtypeStruct((e_pad, 3 * D), jnp.bfloat16),
        mesh=mesh,
        compiler_params=pltpu.CompilerParams(use_tc_tiling_on_sc=False),
        scratch_types=[pltpu.VMEM((CHS, D), jnp.bfloat16)] * 3
                      + [pltpu.VMEM((CHS,), jnp.int32)] * 3
                      + [pltpu.SemaphoreType.DMA],
    )
    return fn(feats, idxs)


def _seg_body(offs_ref, comb_ref, dst_ref, w_ref, b_ref, m_ref, s_ref, r_ref,
              out_ref, cb0, cb1, db0, db1, sc0, sc1, sd0, sd1):
    k = pl.program_id(0)
    s = offs_ref[k]
    e_ = offs_ref[k + 1]
    c_lo = s // CH
    c_hi = (e_ + CH - 1) // CH
    n = c_hi - c_lo
    rows = k * TT + lax.broadcasted_iota(jnp.int32, (TT, CH), 0)

    def cidx(i):
        return jnp.clip(c_lo + i, 0, jnp.maximum(c_hi - 1, 0))

    def start(i, cb, db, sc, sd):
        c = cidx(i)
        pltpu.make_async_copy(comb_ref.at[pl.ds(c * CH, CH)], cb, sc).start()
        pltpu.make_async_copy(dst_ref.at[pl.ds(c * CH, CH)], db, sd).start()

    def wait(cb, db, sc, sd):
        pltpu.make_async_copy(comb_ref.at[pl.ds(0, CH)], cb, sc).wait()
        pltpu.make_async_copy(dst_ref.at[pl.ds(0, CH)], db, sd).wait()

    def compute(i, cb, db, acc0, acc1):
        comb = cb[...]
        z = jnp.dot(comb, w_ref[...], preferred_element_type=jnp.float32)
        eft = jnp.tanh(z + b_ref[...])
        a2 = jnp.dot(eft, m_ref[...], preferred_element_type=jnp.float32)
        a1 = jnp.dot(comb, s_ref[...], preferred_element_type=jnp.float32)
        a = a1 + a2
        a = jnp.where(a >= 0, a, 0.2 * a)
        ea = jnp.exp(a)
        wef = eft * jnp.dot(ea, r_ref[...],
                            preferred_element_type=jnp.float32)
        dstv = jnp.where(i < n, db[...], -1)
        eq = rows == dstv.reshape(1, CH)
        oh = eq.astype(jnp.float32)
        oh16 = eq.astype(jnp.bfloat16)
        acc0 = acc0 + jnp.dot(oh16, wef.astype(jnp.bfloat16),
                              preferred_element_type=jnp.float32)
        acc1 = acc1 + jnp.dot(oh, ea, preferred_element_type=jnp.float32)
        return acc0, acc1

    start(0, cb0, db0, sc0, sd0)
    acc0 = jnp.zeros((TT, HD), jnp.float32)
    acc1 = jnp.zeros((TT, H), jnp.float32)

    def pair(ip, carry):
        acc0, acc1 = carry
        i0 = 2 * ip
        start(i0 + 1, cb1, db1, sc1, sd1)
        wait(cb0, db0, sc0, sd0)
        acc0, acc1 = compute(i0, cb0, db0, acc0, acc1)
        start(i0 + 2, cb0, db0, sc0, sd0)
        wait(cb1, db1, sc1, sd1)
        acc0, acc1 = compute(i0 + 1, cb1, db1, acc0, acc1)
        return acc0, acc1

    acc0, acc1 = lax.fori_loop(0, (n + 1) // 2, pair, (acc0, acc1))
    wait(cb0, db0, sc0, sd0)
    scale = 1.0 / (acc1 + 1e-9)
    ft = acc0 * jnp.dot(scale, r_ref[...], preferred_element_type=jnp.float32)
    out_ref[...] = jnp.where(ft > 0, ft, jnp.exp(jnp.minimum(ft, 0.0)) - 1.0)


def _seg_pass(offs, comb, dst, w, b, m, sel, r, t_pad):
    return pl.pallas_call(
        _seg_body,
        grid=(t_pad // TT,),
        in_specs=[
            pl.BlockSpec(memory_space=pltpu.MemorySpace.SMEM),
            pl.BlockSpec(memory_space=pl.ANY),
            pl.BlockSpec(memory_space=pl.ANY),
            pl.BlockSpec((3 * D, HD), lambda i: (0, 0)),
            pl.BlockSpec((1, HD), lambda i: (0, 0)),
            pl.BlockSpec((HD, H), lambda i: (0, 0)),
            pl.BlockSpec((3 * D, H), lambda i: (0, 0)),
            pl.BlockSpec((H, HD), lambda i: (0, 0)),
        ],
        out_specs=pl.BlockSpec((TT, HD), lambda i: (i, 0)),
        out_shape=jax.ShapeDtypeStruct((t_pad, HD), jnp.float32),
        scratch_shapes=[
            pltpu.VMEM((CH, 3 * D), jnp.bfloat16),
            pltpu.VMEM((CH, 3 * D), jnp.bfloat16),
            pltpu.VMEM((CH,), jnp.int32),
            pltpu.VMEM((CH,), jnp.int32),
            pltpu.SemaphoreType.DMA,
            pltpu.SemaphoreType.DMA,
            pltpu.SemaphoreType.DMA,
            pltpu.SemaphoreType.DMA,
        ],
    )(offs, comb, dst, w, b, m, sel, r)


def kernel(features_0, features_1, type_mask, mp_m0, mp_m1, mp_d0, mp_d1,
           dst_m0, dst_m1, dst_d0, dst_d1, target_m, target_d, params):
    p = params
    t = features_0.shape[0]
    e = mp_m0.shape[0]
    chunk = NW * CHS
    e_pad = ((e + chunk - 1) // chunk) * chunk
    t_pad = ((t + TT - 1) // TT) * TT

    tf0 = features_0 @ p['fc0_W'].T + p['fc0_b']
    tf1 = features_1 @ p['fc1_W'].T + p['fc1_b']
    feats = jnp.concatenate([tf0, tf1], axis=0)

    eye = jnp.eye(H, dtype=jnp.float32)
    rep = jnp.repeat(eye, D, axis=1)

    mps = {'m': (mp_m0, mp_m1), 'd': (mp_d0, mp_d1)}
    dsts = {'m': (dst_m0, dst_m1), 'd': (dst_d0, dst_d1)}
    keys = [('m', 0), ('m', 1), ('d', 0), ('d', 1)]

    feats16 = feats.astype(jnp.bfloat16)
    pad_i = jnp.zeros((e_pad - e, 3), jnp.int32)
    combd = {}
    for nt, i in keys:
        idxs = jnp.concatenate([mps[nt][i], pad_i], axis=0).T.reshape(-1)
        combd[(nt, i)] = _sc_gather(feats16, idxs, e_pad)

    pad_d = jnp.full((e_pad - e,), t, jnp.int32)
    tile_starts = jnp.arange(t_pad // TT + 1, dtype=jnp.int32) * TT
    outs = {}
    for nt in ('m', 'd'):
        res = []
        for i in range(2):
            w = jnp.tile(p[nt + '_rnn_W' + str(i)].T / 3.0, (3, 1)).astype(jnp.bfloat16)
            sel = jnp.zeros((3 * D, H), jnp.float32).at[2 * D:].set(
                p[nt + '_attn1_W' + str(i)].T).astype(jnp.bfloat16)
            b = p[nt + '_rnn_b' + str(i)].reshape(1, HD)
            attn2 = p[nt + '_attn2_' + str(i)]
            m = (attn2[:, :, None] * eye[:, None, :]).reshape(HD, H)
            dst = jnp.concatenate([dsts[nt][i], pad_d])
            offs = jnp.searchsorted(dst, tile_starts).astype(jnp.int32)
            ft = _seg_pass(offs, combd[(nt, i)], dst, w, b, m, sel, rep,
                           t_pad)
            res.append(ft[:t])
        betas = []
        for out in res:
            s = jnp.tanh(out @ p[nt + '_sem_fc1_W'].T + p[nt + '_sem_fc1_b'])
            betas.append(jnp.mean(s @ p[nt + '_sem_fc2_W'].T))
        beta = jax.nn.softmax(jnp.stack(betas))
        outs[nt] = beta[0] * res[0] + beta[1] * res[1]

    lm = outs['m'] @ p['m_out_W'].T + p['m_out_b']
    ld = outs['d'] @ p['d_out_W'].T + p['d_out_b']
    return (lm, ld, outs['m'], outs['d'])


# speedup vs baseline: 1.0041x; 1.0041x over previous
import functools

import jax
import jax.numpy as jnp
from jax import lax
from jax.experimental import pallas as pl
from jax.experimental.pallas import tpu as pltpu
from jax.experimental.pallas import tpu_sc as plsc

H = 8
D = 64
HD = H * D

NC = 2
NS = 16
NW = NC * NS
CHS = 296

CH = 512
TT = 128


def _sc_gather_body(feats_hbm, idx_hbm, out,
                    b0, b1, b2, ib0, ib1, ib2, sem,
                    *, ch, k, e_pad):
    wid = lax.axis_index("s") * NC + lax.axis_index("c")
    ibs = (ib0, ib1, ib2)
    bufs = (b0, b1, b2)

    def jbody(j, _):
        row0 = wid * ch + j * CHS
        for l in range(3):
            pltpu.sync_copy(idx_hbm.at[pl.ds(l * e_pad + row0, CHS)], ibs[l])
        cps = [pltpu.async_copy(feats_hbm.at[ibs[l]], bufs[l], sem)
               for l in range(3)]
        for c in cps:
            c.wait()
        for l in range(3):
            pltpu.sync_copy(bufs[l],
                            out.at[pl.ds(row0, CHS), pl.ds(l * D, D)])
        return 0

    lax.fori_loop(0, k, jbody, 0)


def _sc_gather(feats, idxs, e_pad):
    ch = e_pad // NW
    k = ch // CHS
    mesh = plsc.VectorSubcoreMesh(core_axis_name="c", subcore_axis_name="s")
    fn = pl.kernel(
        functools.partial(_sc_gather_body, ch=ch, k=k, e_pad=e_pad),
        out_type=jax.ShapeDtypeStruct((e_pad, 3 * D), jnp.bfloat16),
        mesh=mesh,
        compiler_params=pltpu.CompilerParams(use_tc_tiling_on_sc=False),
        scratch_types=[pltpu.VMEM((CHS, D), jnp.bfloat16)] * 3
                      + [pltpu.VMEM((CHS,), jnp.int32)] * 3
                      + [pltpu.SemaphoreType.DMA],
    )
    return fn(feats, idxs)


def _seg_body(offs_ref, comb_ref, dst_ref, w_ref, b_ref, m_ref, s_ref, r_ref,
              out_ref, cb0, cb1, db0, db1, sc0, sc1, sd0, sd1):
    k = pl.program_id(0)
    s = offs_ref[k]
    e_ = offs_ref[k + 1]
    c_lo = s // CH
    c_hi = (e_ + CH - 1) // CH
    n = c_hi - c_lo
    rows = k * TT + lax.broadcasted_iota(jnp.int32, (TT, CH), 0)

    def cidx(i):
        return jnp.clip(c_lo + i, 0, jnp.maximum(c_hi - 1, 0))

    def start(i, cb, db, sc, sd):
        c = cidx(i)
        pltpu.make_async_copy(comb_ref.at[pl.ds(c * CH, CH)], cb, sc).start()
        pltpu.make_async_copy(dst_ref.at[pl.ds(c * CH, CH)], db, sd).start()

    def wait(cb, db, sc, sd):
        pltpu.make_async_copy(comb_ref.at[pl.ds(0, CH)], cb, sc).wait()
        pltpu.make_async_copy(dst_ref.at[pl.ds(0, CH)], db, sd).wait()

    def compute(i, cb, db, acc0, acc1):
        comb = cb[...]
        z = jnp.dot(comb, w_ref[...], preferred_element_type=jnp.float32)
        eft = jnp.tanh(z + b_ref[...])
        a2 = jnp.dot(eft, m_ref[...], preferred_element_type=jnp.float32)
        a1 = jnp.dot(comb, s_ref[...], preferred_element_type=jnp.float32)
        a = a1 + a2
        a = jnp.where(a >= 0, a, 0.2 * a)
        ea = jnp.exp(a)
        wef = eft * jnp.dot(ea, r_ref[...],
                            preferred_element_type=jnp.float32)
        dstv = jnp.where(i < n, db[...], -1)
        oh = (rows == dstv.reshape(1, CH)).astype(jnp.float32)
        acc0 = acc0 + jnp.dot(oh, wef, preferred_element_type=jnp.float32)
        acc1 = acc1 + jnp.dot(oh, ea, preferred_element_type=jnp.float32)
        return acc0, acc1

    start(0, cb0, db0, sc0, sd0)
    acc0 = jnp.zeros((TT, HD), jnp.float32)
    acc1 = jnp.zeros((TT, H), jnp.float32)

    def pair(ip, carry):
        acc0, acc1 = carry
        i0 = 2 * ip
        start(i0 + 1, cb1, db1, sc1, sd1)
        wait(cb0, db0, sc0, sd0)
        acc0, acc1 = compute(i0, cb0, db0, acc0, acc1)
        start(i0 + 2, cb0, db0, sc0, sd0)
        wait(cb1, db1, sc1, sd1)
        acc0, acc1 = compute(i0 + 1, cb1, db1, acc0, acc1)
        return acc0, acc1

    acc0, acc1 = lax.fori_loop(0, (n + 1) // 2, pair, (acc0, acc1))
    wait(cb0, db0, sc0, sd0)
    scale = 1.0 / (acc1 + 1e-9)
    ft = acc0 * jnp.dot(scale, r_ref[...], preferred_element_type=jnp.float32)
    out_ref[...] = jnp.where(ft > 0, ft, jnp.exp(jnp.minimum(ft, 0.0)) - 1.0)


def _seg_pass(offs, comb, dst, w, b, m, sel, r, t_pad):
    return pl.pallas_call(
        _seg_body,
        grid=(t_pad // TT,),
        in_specs=[
            pl.BlockSpec(memory_space=pltpu.MemorySpace.SMEM),
            pl.BlockSpec(memory_space=pl.ANY),
            pl.BlockSpec(memory_space=pl.ANY),
            pl.BlockSpec((3 * D, HD), lambda i: (0, 0)),
            pl.BlockSpec((1, HD), lambda i: (0, 0)),
            pl.BlockSpec((HD, H), lambda i: (0, 0)),
            pl.BlockSpec((3 * D, H), lambda i: (0, 0)),
            pl.BlockSpec((H, HD), lambda i: (0, 0)),
        ],
        out_specs=pl.BlockSpec((TT, HD), lambda i: (i, 0)),
        out_shape=jax.ShapeDtypeStruct((t_pad, HD), jnp.float32),
        scratch_shapes=[
            pltpu.VMEM((CH, 3 * D), jnp.bfloat16),
            pltpu.VMEM((CH, 3 * D), jnp.bfloat16),
            pltpu.VMEM((CH,), jnp.int32),
            pltpu.VMEM((CH,), jnp.int32),
            pltpu.SemaphoreType.DMA,
            pltpu.SemaphoreType.DMA,
            pltpu.SemaphoreType.DMA,
            pltpu.SemaphoreType.DMA,
        ],
    )(offs, comb, dst, w, b, m, sel, r)


def kernel(features_0, features_1, type_mask, mp_m0, mp_m1, mp_d0, mp_d1,
           dst_m0, dst_m1, dst_d0, dst_d1, target_m, target_d, params):
    p = params
    t = features_0.shape[0]
    e = mp_m0.shape[0]
    chunk = NW * CHS
    e_pad = ((e + chunk - 1) // chunk) * chunk
    t_pad = ((t + TT - 1) // TT) * TT

    tf0 = features_0 @ p['fc0_W'].T + p['fc0_b']
    tf1 = features_1 @ p['fc1_W'].T + p['fc1_b']
    feats = jnp.concatenate([tf0, tf1], axis=0)

    eye = jnp.eye(H, dtype=jnp.float32)
    rep = jnp.repeat(eye, D, axis=1)

    mps = {'m': (mp_m0, mp_m1), 'd': (mp_d0, mp_d1)}
    dsts = {'m': (dst_m0, dst_m1), 'd': (dst_d0, dst_d1)}
    keys = [('m', 0), ('m', 1), ('d', 0), ('d', 1)]

    feats16 = feats.astype(jnp.bfloat16)
    pad_i = jnp.zeros((e_pad - e, 3), jnp.int32)
    combd = {}
    for nt, i in keys:
        idxs = jnp.concatenate([mps[nt][i], pad_i], axis=0).T.reshape(-1)
        combd[(nt, i)] = _sc_gather(feats16, idxs, e_pad)

    pad_d = jnp.full((e_pad - e,), t, jnp.int32)
    tile_starts = jnp.arange(t_pad // TT + 1, dtype=jnp.int32) * TT
    outs = {}
    for nt in ('m', 'd'):
        res = []
        for i in range(2):
            w = jnp.tile(p[nt + '_rnn_W' + str(i)].T / 3.0, (3, 1)).astype(jnp.bfloat16)
            sel = jnp.zeros((3 * D, H), jnp.float32).at[2 * D:].set(
                p[nt + '_attn1_W' + str(i)].T).astype(jnp.bfloat16)
            b = p[nt + '_rnn_b' + str(i)].reshape(1, HD)
            attn2 = p[nt + '_attn2_' + str(i)]
            m = (attn2[:, :, None] * eye[:, None, :]).reshape(HD, H)
            dst = jnp.concatenate([dsts[nt][i], pad_d])
            offs = jnp.searchsorted(dst, tile_starts).astype(jnp.int32)
            ft = _seg_pass(offs, combd[(nt, i)], dst, w, b, m, sel, rep,
                           t_pad)
            res.append(ft[:t])
        betas = []
        for out in res:
            s = jnp.tanh(out @ p[nt + '_sem_fc1_W'].T + p[nt + '_sem_fc1_b'])
            betas.append(jnp.mean(s @ p[nt + '_sem_fc2_W'].T))
        beta = jax.nn.softmax(jnp.stack(betas))
        outs[nt] = beta[0] * res[0] + beta[1] * res[1]

    lm = outs['m'] @ p['m_out_W'].T + p['m_out_b']
    ld = outs['d'] @ p['d_out_W'].T + p['d_out_b']
    return (lm, ld, outs['m'], outs['d'])


# speedup vs baseline: 1.1933x; 1.1884x over previous
import functools

import jax
import jax.numpy as jnp
from jax import lax
from jax.experimental import pallas as pl
from jax.experimental.pallas import tpu as pltpu
from jax.experimental.pallas import tpu_sc as plsc

H = 8
D = 64
HD = H * D

NC = 2
NS = 16
NW = NC * NS
CHS = 296

CH = 512
TT = 128


def _sc_gather_body(feats_hbm, idx_hbm, out,
                    b0, b1, b2, ib0, ib1, ib2, sem,
                    *, ch, k, e_pad):
    wid = lax.axis_index("s") * NC + lax.axis_index("c")
    ibs = (ib0, ib1, ib2)
    bufs = (b0, b1, b2)

    def jbody(j, _):
        row0 = wid * ch + j * CHS
        for l in range(3):
            pltpu.sync_copy(idx_hbm.at[pl.ds(l * e_pad + row0, CHS)], ibs[l])
        cps = [pltpu.async_copy(feats_hbm.at[ibs[l]], bufs[l], sem)
               for l in range(3)]
        for c in cps:
            c.wait()
        for l in range(3):
            pltpu.sync_copy(bufs[l],
                            out.at[pl.ds(row0, CHS), pl.ds(l * D, D)])
        return 0

    lax.fori_loop(0, k, jbody, 0)


def _sc_gather(feats, idxs, e_pad):
    ch = e_pad // NW
    k = ch // CHS
    mesh = plsc.VectorSubcoreMesh(core_axis_name="c", subcore_axis_name="s")
    fn = pl.kernel(
        functools.partial(_sc_gather_body, ch=ch, k=k, e_pad=e_pad),
        out_type=jax.ShapeDtypeStruct((e_pad, 3 * D), jnp.float32),
        mesh=mesh,
        compiler_params=pltpu.CompilerParams(use_tc_tiling_on_sc=False),
        scratch_types=[pltpu.VMEM((CHS, D), jnp.float32)] * 3
                      + [pltpu.VMEM((CHS,), jnp.int32)] * 3
                      + [pltpu.SemaphoreType.DMA],
    )
    return fn(feats, idxs)


def _seg_body(offs_ref, comb_ref, dst_ref, w_ref, b_ref, m_ref, s_ref, r_ref,
              out_ref, cb0, cb1, db0, db1, sc0, sc1, sd0, sd1, *, nchunks):
    k = pl.program_id(0)
    s = offs_ref[k]
    e_ = offs_ref[k + 1]
    c_lo = s // CH
    c_hi = (e_ + CH - 1) // CH
    n = c_hi - c_lo
    nxt = jnp.minimum(e_ // CH, nchunks - 1)
    rows = k * TT + lax.broadcasted_iota(jnp.int32, (TT, CH), 0)

    def cidx(i):
        return jnp.minimum(c_lo + i, nxt)

    def start(i, cb, db, sc, sd):
        c = cidx(i)
        pltpu.make_async_copy(comb_ref.at[pl.ds(c * CH, CH)], cb, sc).start()
        pltpu.make_async_copy(dst_ref.at[pl.ds(c * CH, CH)], db, sd).start()

    def wait(cb, db, sc, sd):
        pltpu.make_async_copy(comb_ref.at[pl.ds(0, CH)], cb, sc).wait()
        pltpu.make_async_copy(dst_ref.at[pl.ds(0, CH)], db, sd).wait()

    def compute(i, cb, db, acc0, acc1):
        comb = cb[...]
        z = jnp.dot(comb, w_ref[...], preferred_element_type=jnp.float32)
        eft = jnp.tanh(z + b_ref[...])
        a2 = jnp.dot(eft, m_ref[...], preferred_element_type=jnp.float32)
        a1 = jnp.dot(comb, s_ref[...], preferred_element_type=jnp.float32)
        a = a1 + a2
        a = jnp.where(a >= 0, a, 0.2 * a)
        ea = jnp.exp(a)
        wef = eft * jnp.dot(ea, r_ref[...],
                            preferred_element_type=jnp.float32)
        dstv = jnp.where(i < n, db[...], -1)
        oh = (rows == dstv.reshape(1, CH)).astype(jnp.float32)
        acc0 = acc0 + jnp.dot(oh, wef, preferred_element_type=jnp.float32)
        acc1 = acc1 + jnp.dot(oh, ea, preferred_element_type=jnp.float32)
        return acc0, acc1

    @pl.when(k == 0)
    def _prologue():
        start(0, cb0, db0, sc0, sd0)

    acc0 = jnp.zeros((TT, HD), jnp.float32)
    acc1 = jnp.zeros((TT, H), jnp.float32)

    def pair(ip, carry):
        acc0, acc1 = carry
        i0 = 2 * ip
        start(i0 + 1, cb1, db1, sc1, sd1)
        wait(cb0, db0, sc0, sd0)
        acc0, acc1 = compute(i0, cb0, db0, acc0, acc1)
        start(i0 + 2, cb0, db0, sc0, sd0)
        wait(cb1, db1, sc1, sd1)
        acc0, acc1 = compute(i0 + 1, cb1, db1, acc0, acc1)
        return acc0, acc1

    acc0, acc1 = lax.fori_loop(0, (n + 1) // 2, pair, (acc0, acc1))

    @pl.when(k == pl.num_programs(0) - 1)
    def _drain():
        wait(cb0, db0, sc0, sd0)

    scale = 1.0 / (acc1 + 1e-9)
    ft = acc0 * jnp.dot(scale, r_ref[...], preferred_element_type=jnp.float32)
    out_ref[...] = jnp.where(ft > 0, ft, jnp.exp(jnp.minimum(ft, 0.0)) - 1.0)


def _seg_pass(offs, comb, dst, w, b, m, sel, r, t_pad):
    return pl.pallas_call(
        functools.partial(_seg_body, nchunks=comb.shape[0] // CH),
        grid=(t_pad // TT,),
        in_specs=[
            pl.BlockSpec(memory_space=pltpu.MemorySpace.SMEM),
            pl.BlockSpec(memory_space=pl.ANY),
            pl.BlockSpec(memory_space=pl.ANY),
            pl.BlockSpec((3 * D, HD), lambda i: (0, 0)),
            pl.BlockSpec((1, HD), lambda i: (0, 0)),
            pl.BlockSpec((HD, H), lambda i: (0, 0)),
            pl.BlockSpec((3 * D, H), lambda i: (0, 0)),
            pl.BlockSpec((H, HD), lambda i: (0, 0)),
        ],
        out_specs=pl.BlockSpec((TT, HD), lambda i: (i, 0)),
        out_shape=jax.ShapeDtypeStruct((t_pad, HD), jnp.float32),
        scratch_shapes=[
            pltpu.VMEM((CH, 3 * D), jnp.float32),
            pltpu.VMEM((CH, 3 * D), jnp.float32),
            pltpu.VMEM((CH,), jnp.int32),
            pltpu.VMEM((CH,), jnp.int32),
            pltpu.SemaphoreType.DMA,
            pltpu.SemaphoreType.DMA,
            pltpu.SemaphoreType.DMA,
            pltpu.SemaphoreType.DMA,
        ],
    )(offs, comb, dst, w, b, m, sel, r)


def kernel(features_0, features_1, type_mask, mp_m0, mp_m1, mp_d0, mp_d1,
           dst_m0, dst_m1, dst_d0, dst_d1, target_m, target_d, params):
    p = params
    t = features_0.shape[0]
    e = mp_m0.shape[0]
    chunk = NW * CHS
    e_pad = ((e + chunk - 1) // chunk) * chunk
    t_pad = ((t + TT - 1) // TT) * TT

    tf0 = features_0 @ p['fc0_W'].T + p['fc0_b']
    tf1 = features_1 @ p['fc1_W'].T + p['fc1_b']
    feats = jnp.concatenate([tf0, tf1], axis=0)

    eye = jnp.eye(H, dtype=jnp.float32)
    rep = jnp.repeat(eye, D, axis=1)

    mps = {'m': (mp_m0, mp_m1), 'd': (mp_d0, mp_d1)}
    dsts = {'m': (dst_m0, dst_m1), 'd': (dst_d0, dst_d1)}
    keys = [('m', 0), ('m', 1), ('d', 0), ('d', 1)]

    pad_i = jnp.zeros((e_pad - e, 3), jnp.int32)
    combd = {}
    for nt, i in keys:
        idxs = jnp.concatenate([mps[nt][i], pad_i], axis=0).T.reshape(-1)
        combd[(nt, i)] = _sc_gather(feats, idxs, e_pad)

    pad_d = jnp.full((e_pad - e,), t, jnp.int32)
    tile_starts = jnp.arange(t_pad // TT + 1, dtype=jnp.int32) * TT
    outs = {}
    for nt in ('m', 'd'):
        res = []
        for i in range(2):
            w = jnp.tile(p[nt + '_rnn_W' + str(i)].T / 3.0, (3, 1))
            sel = jnp.zeros((3 * D, H), jnp.float32).at[2 * D:].set(
                p[nt + '_attn1_W' + str(i)].T)
            b = p[nt + '_rnn_b' + str(i)].reshape(1, HD)
            attn2 = p[nt + '_attn2_' + str(i)]
            m = (attn2[:, :, None] * eye[:, None, :]).reshape(HD, H)
            dst = jnp.concatenate([dsts[nt][i], pad_d])
            offs = jnp.searchsorted(dst, tile_starts).astype(jnp.int32)
            ft = _seg_pass(offs, combd[(nt, i)], dst, w, b, m, sel, rep,
                           t_pad)
            res.append(ft[:t])
        betas = []
        for out in res:
            s = jnp.tanh(out @ p[nt + '_sem_fc1_W'].T + p[nt + '_sem_fc1_b'])
            betas.append(jnp.mean(s @ p[nt + '_sem_fc2_W'].T))
        beta = jax.nn.softmax(jnp.stack(betas))
        outs[nt] = beta[0] * res[0] + beta[1] * res[1]

    lm = outs['m'] @ p['m_out_W'].T + p['m_out_b']
    ld = outs['d'] @ p['d_out_W'].T + p['d_out_b']
    return (lm, ld, outs['m'], outs['d'])
